# baseline (speedup 1.0000x reference)
import jax
import jax.numpy as jnp
from jax.experimental import pallas as pl


def _add_kernel(x_ref, emb_ref, o_ref):
    o_ref[...] = x_ref[...] + emb_ref[...]


def kernel(x, embed_weight):
    B, W, F = x.shape
    emb = embed_weight[:W]
    BLK = 2048
    nj = W // BLK
    return pl.pallas_call(
        _add_kernel,
        grid=(nj, B),
        in_specs=[
            pl.BlockSpec((1, BLK, F), lambda j, b: (b, j, 0)),
            pl.BlockSpec((BLK, F), lambda j, b: (j, 0)),
        ],
        out_specs=pl.BlockSpec((1, BLK, F), lambda j, b: (b, j, 0)),
        out_shape=jax.ShapeDtypeStruct(x.shape, x.dtype),
    )(x, emb)


# speedup vs baseline: 1.5282x; 1.5282x over previous
import jax
import jax.numpy as jnp
from jax.experimental import pallas as pl


def _add_kernel(x_ref, emb_ref, o_ref):
    o_ref[...] = x_ref[...] + emb_ref[...]


def kernel(x, embed_weight):
    B, W, F = x.shape
    emb = embed_weight[:W]
    return pl.pallas_call(
        _add_kernel,
        grid=(B,),
        in_specs=[
            pl.BlockSpec((1, W, F), lambda b: (b, 0, 0)),
            pl.BlockSpec((W, F), lambda b: (0, 0)),
        ],
        out_specs=pl.BlockSpec((1, W, F), lambda b: (b, 0, 0)),
        out_shape=jax.ShapeDtypeStruct(x.shape, x.dtype),
    )(x, emb)


# speedup vs baseline: 1.5643x; 1.0236x over previous
import jax
import jax.numpy as jnp
from jax.experimental import pallas as pl


def _add_kernel(x_ref, emb_ref, o_ref):
    o_ref[...] = x_ref[...] + emb_ref[...]


def kernel(x, embed_weight):
    B, W, F = x.shape
    emb = embed_weight[:W]
    BB = 2
    return pl.pallas_call(
        _add_kernel,
        grid=(B // BB,),
        in_specs=[
            pl.BlockSpec((BB, W, F), lambda b: (b, 0, 0)),
            pl.BlockSpec((W, F), lambda b: (0, 0)),
        ],
        out_specs=pl.BlockSpec((BB, W, F), lambda b: (b, 0, 0)),
        out_shape=jax.ShapeDtypeStruct(x.shape, x.dtype),
    )(x, emb)


# speedup vs baseline: 1.5646x; 1.0002x over previous
import jax
import jax.numpy as jnp
from jax.experimental import pallas as pl
from jax.experimental.pallas import tpu as pltpu


def _add_kernel(x_ref, emb_ref, o_ref):
    o_ref[...] = x_ref[...] + emb_ref[...]


def kernel(x, embed_weight):
    B, W, F = x.shape
    emb = embed_weight[:W]
    BB = 2
    return pl.pallas_call(
        _add_kernel,
        grid=(B // BB,),
        in_specs=[
            pl.BlockSpec((BB, W, F), lambda b: (b, 0, 0)),
            pl.BlockSpec((W, F), lambda b: (0, 0)),
        ],
        out_specs=pl.BlockSpec((BB, W, F), lambda b: (b, 0, 0)),
        out_shape=jax.ShapeDtypeStruct(x.shape, x.dtype),
        compiler_params=pltpu.CompilerParams(
            vmem_limit_bytes=100 * 1024 * 1024,
        ),
    )(x, emb)
